# baseline (speedup 1.0000x reference)
import jax
import jax.numpy as jnp
from jax import lax
from jax.experimental import pallas as pl
from jax.experimental.pallas import tpu as pltpu
from jax.experimental.pallas import tpu_sc as plsc

DIM0 = 100000
EMB_DIM = 128
N = 16384

NUM_CORES = 2
NUM_SUBCORES = 16
NW = NUM_CORES * NUM_SUBCORES
ROWS_PER_W = N // NW
CHUNK = 64
NCHUNK = ROWS_PER_W // CHUNK


def _sc_kernel(xc_hbm, emb0_hbm, emb1_hbm, out_hbm,
               idx_v, bufv, gsem, ssem):
    wid = lax.axis_index("s") * NUM_CORES + lax.axis_index("c")
    base = wid * ROWS_PER_W

    pltpu.sync_copy(xc_hbm.at[wid], idx_v)

    cp0 = [None] * NCHUNK
    cp1 = [None] * NCHUNK
    st = [None] * NCHUNK

    def g0(j):
        return pltpu.async_copy(emb0_hbm.at[idx_v.at[0, j]], bufv.at[j],
                                gsem.at[j])

    cp0[0] = g0(0)
    cp0[1] = g0(1)
    for j in range(NCHUNK):
        cp0[j].wait()
        cp1[j] = pltpu.async_copy(emb1_hbm.at[idx_v.at[1, j]], bufv.at[j],
                                  gsem.at[j], add=True)
        if j + 2 < NCHUNK:
            cp0[j + 2] = g0(j + 2)
        if j >= 1:
            cp1[j - 1].wait()
            st[j - 1] = pltpu.async_copy(
                bufv.at[j - 1],
                out_hbm.at[pl.ds(base + (j - 1) * CHUNK, CHUNK)],
                ssem.at[j - 1])
    cp1[NCHUNK - 1].wait()
    st[NCHUNK - 1] = pltpu.async_copy(
        bufv.at[NCHUNK - 1],
        out_hbm.at[pl.ds(base + (NCHUNK - 1) * CHUNK, CHUNK)],
        ssem.at[NCHUNK - 1])
    for j in range(NCHUNK):
        st[j].wait()


def kernel(x, emb0, emb1):
    x = x.astype(jnp.int32)
    xc = x.reshape(NW, NCHUNK, CHUNK, 2).transpose(0, 3, 1, 2)

    mesh = plsc.VectorSubcoreMesh(core_axis_name="c", subcore_axis_name="s")
    run = pl.kernel(
        _sc_kernel,
        mesh=mesh,
        out_type=jax.ShapeDtypeStruct((N, EMB_DIM), jnp.float32),
        scratch_types=[
            pltpu.VMEM((2, NCHUNK, CHUNK), jnp.int32),
            pltpu.VMEM((NCHUNK, CHUNK, EMB_DIM), jnp.float32),
            pltpu.SemaphoreType.DMA((NCHUNK,)),
            pltpu.SemaphoreType.DMA((NCHUNK,)),
        ],
    )
    return run(xc, emb0, emb1)


# speedup vs baseline: 1.0408x; 1.0408x over previous
import jax
import jax.numpy as jnp
from jax import lax
from jax.experimental import pallas as pl
from jax.experimental.pallas import tpu as pltpu
from jax.experimental.pallas import tpu_sc as plsc

DIM0 = 100000
EMB_DIM = 128
N = 16384

NUM_CORES = 2
NUM_SUBCORES = 16
NW = NUM_CORES * NUM_SUBCORES
ROWS_PER_W = N // NW
CHUNK = 64
NCHUNK = ROWS_PER_W // CHUNK


def _sc_kernel(xc_hbm, emb0_hbm, emb1_hbm, out_hbm,
               idx_v, bufv, gsem, ssem):
    wid = lax.axis_index("s") * NUM_CORES + lax.axis_index("c")
    base = wid * ROWS_PER_W

    pltpu.sync_copy(xc_hbm.at[wid], idx_v)

    cp0 = [pltpu.async_copy(emb0_hbm.at[idx_v.at[0, j]], bufv.at[j],
                            gsem.at[j])
           for j in range(NCHUNK)]
    cp1 = [None] * NCHUNK
    for j in range(NCHUNK):
        cp0[j].wait()
        cp1[j] = pltpu.async_copy(emb1_hbm.at[idx_v.at[1, j]], bufv.at[j],
                                  gsem.at[j], add=True)
    st = [None] * NCHUNK
    for j in range(NCHUNK):
        cp1[j].wait()
        st[j] = pltpu.async_copy(
            bufv.at[j], out_hbm.at[pl.ds(base + j * CHUNK, CHUNK)],
            ssem.at[j])
    for j in range(NCHUNK):
        st[j].wait()


def kernel(x, emb0, emb1):
    x = x.astype(jnp.int32)
    xc = x.reshape(NW, NCHUNK, CHUNK, 2).transpose(0, 3, 1, 2)

    mesh = plsc.VectorSubcoreMesh(core_axis_name="c", subcore_axis_name="s")
    run = pl.kernel(
        _sc_kernel,
        mesh=mesh,
        out_type=jax.ShapeDtypeStruct((N, EMB_DIM), jnp.float32),
        scratch_types=[
            pltpu.VMEM((2, NCHUNK, CHUNK), jnp.int32),
            pltpu.VMEM((NCHUNK, CHUNK, EMB_DIM), jnp.float32),
            pltpu.SemaphoreType.DMA((NCHUNK,)),
            pltpu.SemaphoreType.DMA((NCHUNK,)),
        ],
    )
    return run(xc, emb0, emb1)
